# baseline (speedup 1.0000x reference)
import functools

import jax
import jax.numpy as jnp
from jax import lax
from jax.experimental import pallas as pl
from jax.experimental.pallas import tpu as pltpu
from jax.experimental.pallas import tpu_sc as plsc

DIM = 768
LANES = 16
NJ = DIM // LANES
EPS = 1e-12
CHUNK = 64


def _lane_shuffle(x, perm):
    dnums = lax.GatherDimensionNumbers(
        offset_dims=(), collapsed_slice_dims=(0,), start_index_map=(0,))
    return lax.gather(x, perm[:, None], dnums, (1,),
                      mode=lax.GatherScatterMode.PROMISE_IN_BOUNDS)


def _lane_total(x):
    for k in (8, 4, 2, 1):
        perm = lax.iota(jnp.int32, LANES) ^ k
        x = x + _lane_shuffle(x, perm)
    return x


def _rsqrt_newton(v):
    i = lax.bitcast_convert_type(v, jnp.int32)
    i = jnp.full((LANES,), 0x5F3759DF, jnp.int32) - lax.shift_right_logical(i, 1)
    y = lax.bitcast_convert_type(i, jnp.float32)
    for _ in range(4):
        y = y * (1.5 - 0.5 * v * y * y)
    return y


@functools.lru_cache(maxsize=None)
def _build(n_tok, seq_len):
    info = plsc.get_sparse_core_info()
    nc, ns = info.num_cores, info.num_subcores
    nw = nc * ns
    per_w = n_tok // nw
    n_chunks = per_w // CHUNK
    assert per_w * nw == n_tok and n_chunks * CHUNK == per_w

    mesh = plsc.VectorSubcoreMesh(core_axis_name="c", subcore_axis_name="s")

    @functools.partial(
        pl.kernel,
        mesh=mesh,
        out_type=jax.ShapeDtypeStruct((n_tok, DIM), jnp.float32),
        scratch_types=[
            pltpu.VMEM((CHUNK,), jnp.int32),
            pltpu.VMEM((CHUNK, DIM), jnp.float32),
            pltpu.VMEM((CHUNK, DIM), jnp.float32),
            pltpu.VMEM((DIM,), jnp.float32),
            pltpu.VMEM((DIM,), jnp.float32),
            pltpu.SemaphoreType.DMA,
        ],
    )
    def sc_kernel(ids_hbm, wemb_hbm, pemb_hbm, gamma_hbm, beta_hbm, out_hbm,
                  idx_v, x_v, pe_v, g_v, b_v, sem):
        wid = lax.axis_index("s") * nc + lax.axis_index("c")
        base = wid * per_w
        pltpu.sync_copy(gamma_hbm, g_v)
        pltpu.sync_copy(beta_hbm, b_v)

        def chunk_body(c, carry):
            tok0 = base + c * CHUNK
            pos0 = lax.rem(tok0, seq_len)
            pltpu.sync_copy(ids_hbm.at[pl.ds(tok0, CHUNK)], idx_v)
            gather = pltpu.async_copy(wemb_hbm.at[idx_v], x_v, sem)
            pltpu.sync_copy(pemb_hbm.at[pl.ds(pos0, CHUNK)], pe_v)
            gather.wait()

            def tok_body(t, carry2):
                s = jnp.zeros((LANES,), jnp.float32)
                ss = jnp.zeros((LANES,), jnp.float32)
                for j in range(NJ):
                    sl = pl.ds(j * LANES, LANES)
                    x = x_v[t, sl] + pe_v[t, sl]
                    x_v[t, sl] = x
                    s = s + x
                    ss = ss + x * x
                mvec = _lane_total(s) * (1.0 / DIM)
                vvec = _lane_total(ss) * (1.0 / DIM) - mvec * mvec + EPS
                r = _rsqrt_newton(vvec)
                for j in range(NJ):
                    sl = pl.ds(j * LANES, LANES)
                    xn = (x_v[t, sl] - mvec) * r
                    x_v[t, sl] = xn * g_v[sl] + b_v[sl]
                return carry2

            lax.fori_loop(0, CHUNK, tok_body, 0)
            pltpu.sync_copy(x_v, out_hbm.at[pl.ds(tok0, CHUNK)])
            return carry

        lax.fori_loop(0, n_chunks, chunk_body, 0)

    return sc_kernel


def kernel(input_ids, word_emb, pos_emb, gamma, beta):
    b, s = input_ids.shape
    ids_flat = input_ids.reshape(-1).astype(jnp.int32)
    out = _build(b * s, s)(ids_flat, word_emb, pos_emb, gamma, beta)
    return out.reshape(b, s, DIM)


# speedup vs baseline: 1.2090x; 1.2090x over previous
import functools

import jax
import jax.numpy as jnp
from jax import lax
from jax.experimental import pallas as pl
from jax.experimental.pallas import tpu as pltpu
from jax.experimental.pallas import tpu_sc as plsc

DIM = 768
LANES = 16
NJ = DIM // LANES
EPS = 1e-12
CHUNK = 32


def _lane_shuffle(x, perm):
    dnums = lax.GatherDimensionNumbers(
        offset_dims=(), collapsed_slice_dims=(0,), start_index_map=(0,))
    return lax.gather(x, perm[:, None], dnums, (1,),
                      mode=lax.GatherScatterMode.PROMISE_IN_BOUNDS)


def _lane_total(x):
    for k in (8, 4, 2, 1):
        perm = lax.iota(jnp.int32, LANES) ^ k
        x = x + _lane_shuffle(x, perm)
    return x


def _rsqrt_newton(v):
    i = lax.bitcast_convert_type(v, jnp.int32)
    i = jnp.full((LANES,), 0x5F3759DF, jnp.int32) - lax.shift_right_logical(i, 1)
    y = lax.bitcast_convert_type(i, jnp.float32)
    for _ in range(4):
        y = y * (1.5 - 0.5 * v * y * y)
    return y


@functools.lru_cache(maxsize=None)
def _build(n_tok, seq_len):
    info = plsc.get_sparse_core_info()
    nc, ns = info.num_cores, info.num_subcores
    nw = nc * ns
    per_w = n_tok // nw
    n_chunks = per_w // CHUNK
    assert per_w * nw == n_tok and n_chunks * CHUNK == per_w and n_chunks % 2 == 0

    mesh = plsc.VectorSubcoreMesh(core_axis_name="c", subcore_axis_name="s")

    @functools.partial(
        pl.kernel,
        mesh=mesh,
        out_type=jax.ShapeDtypeStruct((n_tok, DIM), jnp.float32),
        scratch_types=[
            pltpu.VMEM((2, CHUNK), jnp.int32),
            pltpu.VMEM((CHUNK, DIM), jnp.float32),
            pltpu.VMEM((CHUNK, DIM), jnp.float32),
            pltpu.VMEM((CHUNK, DIM), jnp.float32),
            pltpu.VMEM((CHUNK, DIM), jnp.float32),
            pltpu.VMEM((CHUNK // 2, DIM), jnp.float32),
            pltpu.VMEM((CHUNK, LANES), jnp.float32),
            pltpu.VMEM((CHUNK, LANES), jnp.float32),
            pltpu.VMEM((DIM,), jnp.float32),
            pltpu.VMEM((DIM,), jnp.float32),
            pltpu.SemaphoreType.DMA,
            pltpu.SemaphoreType.DMA,
            pltpu.SemaphoreType.DMA,
        ],
    )
    def sc_kernel(ids_hbm, wemb_hbm, pemb_hbm, gamma_hbm, beta_hbm, out_hbm,
                  idx_v, x0_v, x1_v, pe0_v, pe1_v, o_v, r_v, c_v,
                  g_v, b_v, gsem0, gsem1, osem):
        wid = lax.axis_index("s") * nc + lax.axis_index("c")
        base = wid * per_w
        pltpu.sync_copy(gamma_hbm, g_v)
        pltpu.sync_copy(beta_hbm, b_v)

        xs = (x0_v, x1_v)
        pes = (pe0_v, pe1_v)
        gsems = (gsem0, gsem1)

        def issue(c, p):
            tok0 = base + c * CHUNK
            pos0 = lax.rem(tok0, seq_len)
            pltpu.sync_copy(ids_hbm.at[pl.ds(tok0, CHUNK)], idx_v.at[p])
            pltpu.async_copy(wemb_hbm.at[idx_v.at[p]], xs[p], gsems[p])
            pltpu.async_copy(pemb_hbm.at[pl.ds(pos0, CHUNK)], pes[p], gsems[p])

        def drain_gather(p):
            pltpu.make_async_copy(wemb_hbm.at[idx_v.at[p]], xs[p], gsems[p]).wait()
            pltpu.make_async_copy(pemb_hbm.at[pl.ds(0, CHUNK)], pes[p], gsems[p]).wait()

        HALF = CHUNK // 2

        def drain_out():
            pltpu.make_async_copy(o_v, out_hbm.at[pl.ds(base, HALF)], osem).wait()

        def pass1(x_v, pe_v):
            def tok_body(t, carry):
                s0 = jnp.zeros((LANES,), jnp.float32)
                s1 = jnp.zeros((LANES,), jnp.float32)
                q0 = jnp.zeros((LANES,), jnp.float32)
                q1 = jnp.zeros((LANES,), jnp.float32)
                for j in range(NJ):
                    sl = pl.ds(j * LANES, LANES)
                    x = x_v[t, sl] + pe_v[t, sl]
                    x_v[t, sl] = x
                    if j % 2 == 0:
                        s0 = s0 + x
                        q0 = q0 + x * x
                    else:
                        s1 = s1 + x
                        q1 = q1 + x * x
                mvec = _lane_total(s0 + s1) * (1.0 / DIM)
                vvec = _lane_total(q0 + q1) * (1.0 / DIM) - mvec * mvec + EPS
                r = _rsqrt_newton(vvec)
                r_v[t, :] = r
                c_v[t, :] = -mvec * r
                return carry
            lax.fori_loop(0, CHUNK, tok_body, 0, unroll=2)

        def pass2_half(x_v, h):
            for gh in range(2):
                j0 = gh * (NJ // 2)
                g_regs = [g_v[pl.ds((j0 + j) * LANES, LANES)] for j in range(NJ // 2)]
                b_regs = [b_v[pl.ds((j0 + j) * LANES, LANES)] for j in range(NJ // 2)]

                def tok_body(tl, carry):
                    t = h * HALF + tl
                    a = r_v[t, :]
                    cc = c_v[t, :]
                    for j in range(NJ // 2):
                        sl = pl.ds((j0 + j) * LANES, LANES)
                        o_v[tl, sl] = (x_v[t, sl] * a + cc) * g_regs[j] + b_regs[j]
                    return carry
                lax.fori_loop(0, HALF, tok_body, 0, unroll=2)

        issue(0, 0)

        def pair_body(i2, carry):
            c0 = 2 * i2
            for p in range(2):
                c = c0 + p

                @pl.when(c + 1 < n_chunks)
                def _():
                    issue(c + 1, 1 - p)

                drain_gather(p)
                pass1(xs[p], pes[p])

                for h in range(2):
                    if h == 0:
                        @pl.when(c > 0)
                        def _():
                            drain_out()
                    else:
                        drain_out()
                    pass2_half(xs[p], h)
                    pltpu.async_copy(
                        o_v, out_hbm.at[pl.ds(base + c * CHUNK + h * HALF, HALF)],
                        osem)
            return carry

        lax.fori_loop(0, n_chunks // 2, pair_body, 0)
        drain_out()

    return sc_kernel


def kernel(input_ids, word_emb, pos_emb, gamma, beta):
    b, s = input_ids.shape
    ids_flat = input_ids.reshape(-1).astype(jnp.int32)
    out = _build(b * s, s)(ids_flat, word_emb, pos_emb, gamma, beta)
    return out.reshape(b, s, DIM)
